# Initial kernel scaffold; baseline (speedup 1.0000x reference)
import jax
import jax.numpy as jnp
from jax.experimental import pallas as pl


def kernel(x, W1, root1, b1, W2, root2, b2, Wm1, bm1, Wm2, bm2, edge_index, edge_type, batch):
    raise NotImplementedError("write your pallas kernel here")


# speedup vs baseline: 18.6493x; 18.6493x over previous
import jax
import jax.numpy as jnp
from jax import lax
from jax.experimental import pallas as pl
from jax.experimental.pallas import tpu as pltpu
from jax.experimental.pallas import tpu_sc as plsc

N = 10000
E = 320000
R = 8
G = 32
D = 128
OUT = 8
RW = R + 1

NC = 2
NS = 16
NW = NC * NS
L = 16

EB = 128
NBAT = E // EB
NITER = (NBAT + NW - 1) // NW
CPT = 81920

EPT = E // NW
CH = 400

BN = 1000
NBLK = N // BN

_mesh = plsc.VectorSubcoreMesh(core_axis_name="c", subcore_axis_name="s")


def _worker_ids():
    c = lax.axis_index("c")
    s = lax.axis_index("s")
    return c, s, c * NS + s


def _count_body(dst_ref, et_ref, out_ref, cnt_v, dbuf, ebuf):
    _, _, w = _worker_ids()

    def zero(i, carry):
        cnt_v[pl.ds(i * L, L)] = jnp.zeros((L,), jnp.float32)
        return carry

    lax.fori_loop(0, CPT // L, zero, 0)

    ones = jnp.ones((L,), jnp.float32)

    def chunk(ci, carry):
        base = w * EPT + ci * CH
        pltpu.sync_copy(dst_ref.at[pl.ds(base, CH)], dbuf)
        pltpu.sync_copy(et_ref.at[pl.ds(base, CH)], ebuf)

        def vec(j, c2):
            d16 = dbuf[pl.ds(j * L, L)]
            e16 = ebuf[pl.ds(j * L, L)]
            plsc.addupdate_scatter(cnt_v, [d16 * R + e16], ones)
            return c2

        lax.fori_loop(0, CH // L, vec, 0)
        return carry

    lax.fori_loop(0, EPT // CH, chunk, 0)
    pltpu.sync_copy(cnt_v, out_ref.at[w])


def _k1a(dst, edge_type):
    f = pl.kernel(
        _count_body,
        out_type=jax.ShapeDtypeStruct((NW, CPT), jnp.float32),
        mesh=_mesh,
        compiler_params=pltpu.CompilerParams(needs_layout_passes=False),
        scratch_types=[
            pltpu.VMEM((CPT,), jnp.float32),
            pltpu.VMEM((CH,), jnp.int32),
            pltpu.VMEM((CH,), jnp.int32),
        ],
    )
    return f(dst, edge_type)


def _inv_body(cnt_ref, out_ref):
    s = jnp.sum(cnt_ref[...], axis=0)
    out_ref[...] = 1.0 / jnp.maximum(s, 1.0)


def _k1c(cnt_parts):
    cb = CPT // 128 // 10
    return pl.pallas_call(
        _inv_body,
        grid=(10,),
        in_specs=[pl.BlockSpec((NW, cb, 128), lambda i: (0, i, 0))],
        out_specs=pl.BlockSpec((cb, 128), lambda i: (i, 0)),
        out_shape=jax.ShapeDtypeStruct((CPT // 128, 128), jnp.float32),
    )(cnt_parts)


def _meta_body(src_ref, dst_ref, et_ref, inv_ref, gidx_ref, norm_ref,
               sbuf, dbuf, ebuf, linb, gbuf, cbuf, sem):
    _, _, w = _worker_ids()

    def body(i, carry):
        b = i * NW + w

        @pl.when(b < NBAT)
        def _():
            base = b * EB
            pltpu.sync_copy(src_ref.at[pl.ds(base, EB)], sbuf)
            pltpu.sync_copy(dst_ref.at[pl.ds(base, EB)], dbuf)
            pltpu.sync_copy(et_ref.at[pl.ds(base, EB)], ebuf)

            def vec(j, c2):
                s16 = sbuf[pl.ds(j * L, L)]
                d16 = dbuf[pl.ds(j * L, L)]
                e16 = ebuf[pl.ds(j * L, L)]
                linb[pl.ds(j * L, L)] = d16 * R + e16
                gbuf[pl.ds(j * L, L)] = e16 * N + s16
                return c2

            lax.fori_loop(0, EB // L, vec, 0)
            pltpu.async_copy(inv_ref.at[linb], cbuf, sem).wait()
            pltpu.sync_copy(gbuf, gidx_ref.at[pl.ds(base, EB)])
            pltpu.sync_copy(cbuf, norm_ref.at[pl.ds(base, EB)])

        return carry

    lax.fori_loop(0, NITER, body, 0)


def _k1b(src, dst, edge_type, inv):
    f = pl.kernel(
        _meta_body,
        out_type=(
            jax.ShapeDtypeStruct((E,), jnp.int32),
            jax.ShapeDtypeStruct((E,), jnp.float32),
        ),
        mesh=_mesh,
        compiler_params=pltpu.CompilerParams(needs_layout_passes=False),
        scratch_types=[
            pltpu.VMEM((EB,), jnp.int32),
            pltpu.VMEM((EB,), jnp.int32),
            pltpu.VMEM((EB,), jnp.int32),
            pltpu.VMEM((EB,), jnp.int32),
            pltpu.VMEM((EB,), jnp.int32),
            pltpu.VMEM((EB,), jnp.float32),
            pltpu.SemaphoreType.DMA,
        ],
    )
    return f(src, dst, edge_type, inv)


def _xmm_body(x_ref, w_ref, out_ref):
    out_ref[...] = jnp.dot(x_ref[...], w_ref[0],
                           preferred_element_type=jnp.float32)


def _k2(x, Ws):
    return pl.pallas_call(
        _xmm_body,
        grid=(NBLK, RW),
        in_specs=[
            pl.BlockSpec((BN, D), lambda nb, r: (nb, 0)),
            pl.BlockSpec((1, D, D), lambda nb, r: (r, 0, 0)),
        ],
        out_specs=pl.BlockSpec((BN, D), lambda nb, r: (r * NBLK + nb, 0)),
        out_shape=jax.ShapeDtypeStruct((RW * N, D), jnp.float32),
        compiler_params=pltpu.CompilerParams(
            dimension_semantics=("arbitrary", "arbitrary")),
    )(x, Ws)


def _agg_body(xall_ref, gidx_ref, norm_ref, dst_ref, zeros_ref, out_ref,
              acc, gi, nb, db, rows, sem):
    c, s, w = _worker_ids()

    @pl.when(s == 0)
    def _():
        pltpu.sync_copy(zeros_ref, acc)

    plsc.subcore_barrier()

    def body(i, carry):
        b = i * NW + w

        @pl.when(b < NBAT)
        def _():
            base = b * EB
            pltpu.sync_copy(gidx_ref.at[pl.ds(base, EB)], gi)
            pltpu.sync_copy(norm_ref.at[pl.ds(base, EB)], nb)
            pltpu.sync_copy(dst_ref.at[pl.ds(base, EB)], db)
            pltpu.async_copy(xall_ref.at[gi], rows, sem).wait()

            def scale(j, c2):
                nv = nb[pl.ds(j * L, L)]
                for k in range(L):
                    e = j * L + k
                    sc = nv[k]
                    for f in range(D // L):
                        rows[e, pl.ds(f * L, L)] = (
                            rows[e, pl.ds(f * L, L)] * sc)
                return c2

            lax.fori_loop(0, EB // L, scale, 0)
            pltpu.sync_copy(rows, acc.at[db], add=True)

        return carry

    lax.fori_loop(0, NITER, body, 0)
    plsc.subcore_barrier()

    @pl.when(s < NBLK)
    def _():
        pltpu.sync_copy(acc.at[pl.ds(s * BN, BN)],
                        out_ref.at[c, pl.ds(s * BN, BN)])


def _agg(xall, gidx, norm, dst, zeros_nd):
    f = pl.kernel(
        _agg_body,
        out_type=jax.ShapeDtypeStruct((NC, N, D), jnp.float32),
        mesh=_mesh,
        compiler_params=pltpu.CompilerParams(needs_layout_passes=False),
        scratch_types=[
            pltpu.VMEM_SHARED((N, D), jnp.float32),
            pltpu.VMEM((EB,), jnp.int32),
            pltpu.VMEM((EB,), jnp.float32),
            pltpu.VMEM((EB,), jnp.int32),
            pltpu.VMEM((EB, D), jnp.float32),
            pltpu.SemaphoreType.DMA,
        ],
    )
    return f(xall, gidx, norm, dst, zeros_nd)


def _l2_body(msg_ref, root_ref, b_ref, w_ref, out_ref, h_scr):
    r = pl.program_id(1)

    @pl.when(r == 0)
    def _():
        h_scr[...] = jnp.maximum(
            msg_ref[0] + msg_ref[1] + root_ref[...] + b_ref[...], 0.0)

    out_ref[...] = jnp.dot(h_scr[...], w_ref[0],
                           preferred_element_type=jnp.float32)


def _k4(msg1, xall1, b1r, W2s):
    return pl.pallas_call(
        _l2_body,
        grid=(NBLK, RW),
        in_specs=[
            pl.BlockSpec((NC, BN, D), lambda nb, r: (0, nb, 0)),
            pl.BlockSpec((BN, D), lambda nb, r: (R * NBLK + nb, 0)),
            pl.BlockSpec((1, D), lambda nb, r: (0, 0)),
            pl.BlockSpec((1, D, D), lambda nb, r: (r, 0, 0)),
        ],
        out_specs=pl.BlockSpec((BN, D), lambda nb, r: (r * NBLK + nb, 0)),
        out_shape=jax.ShapeDtypeStruct((RW * N, D), jnp.float32),
        scratch_shapes=[pltpu.VMEM((BN, D), jnp.float32)],
        compiler_params=pltpu.CompilerParams(
            dimension_semantics=("arbitrary", "arbitrary")),
    )(msg1, xall1, b1r, W2s)


def _fin_body(msg_ref, root_ref, b_ref, batch_ref, emb_ref, sums_ref,
              cnts_ref):
    nb = pl.program_id(0)
    emb = jnp.maximum(
        msg_ref[0] + msg_ref[1] + root_ref[...] + b_ref[...], 0.0)
    emb_ref[...] = emb
    bt = batch_ref[0, 0, :]
    onehot = (bt[None, :] ==
              lax.broadcasted_iota(jnp.int32, (G, BN), 0)).astype(jnp.float32)
    ps = jnp.dot(onehot, emb, preferred_element_type=jnp.float32)
    pc = jnp.sum(onehot, axis=1, keepdims=True) * jnp.ones((1, D), jnp.float32)

    @pl.when(nb == 0)
    def _():
        sums_ref[...] = jnp.zeros_like(sums_ref)
        cnts_ref[...] = jnp.zeros_like(cnts_ref)

    sums_ref[...] += ps
    cnts_ref[...] += pc


def _k6(msg2, xall2, b2r, batch3):
    return pl.pallas_call(
        _fin_body,
        grid=(NBLK,),
        in_specs=[
            pl.BlockSpec((NC, BN, D), lambda nb: (0, nb, 0)),
            pl.BlockSpec((BN, D), lambda nb: (R * NBLK + nb, 0)),
            pl.BlockSpec((1, D), lambda nb: (0, 0)),
            pl.BlockSpec((1, 1, BN), lambda nb: (nb, 0, 0)),
        ],
        out_specs=[
            pl.BlockSpec((BN, D), lambda nb: (nb, 0)),
            pl.BlockSpec((G, D), lambda nb: (0, 0)),
            pl.BlockSpec((G, D), lambda nb: (0, 0)),
        ],
        out_shape=[
            jax.ShapeDtypeStruct((N, D), jnp.float32),
            jax.ShapeDtypeStruct((G, D), jnp.float32),
            jax.ShapeDtypeStruct((G, D), jnp.float32),
        ],
        compiler_params=pltpu.CompilerParams(
            dimension_semantics=("arbitrary",)),
    )(msg2, xall2, b2r, batch3)


def _mlp_body(sums_ref, cnts_ref, wm1_ref, bm1_ref, wm2_ref, bm2_ref,
              logits_ref, probs_ref):
    pooled = sums_ref[...] / jnp.maximum(cnts_ref[...], 1.0)
    v = jnp.dot(pooled, wm1_ref[...],
                preferred_element_type=jnp.float32) + bm1_ref[...]
    hm = jnp.where(v > 0, v, jnp.exp(jnp.minimum(v, 0.0)) - 1.0)
    lg = jnp.dot(hm, wm2_ref[...],
                 preferred_element_type=jnp.float32) + bm2_ref[...]
    logits_ref[...] = lg
    colmask = lax.broadcasted_iota(jnp.int32, (G, D), 1) < OUT
    lgm = jnp.where(colmask, lg, -1e30)
    m = jnp.max(lgm, axis=1, keepdims=True)
    ex = jnp.where(colmask, jnp.exp(lgm - m), 0.0)
    probs_ref[...] = ex / jnp.sum(ex, axis=1, keepdims=True)


def _k7(sums, cnts, Wm1, bm1r, Wm2p, bm2p):
    return pl.pallas_call(
        _mlp_body,
        out_shape=[
            jax.ShapeDtypeStruct((G, D), jnp.float32),
            jax.ShapeDtypeStruct((G, D), jnp.float32),
        ],
    )(sums, cnts, Wm1, bm1r, Wm2p, bm2p)


def kernel(x, W1, root1, b1, W2, root2, b2, Wm1, bm1, Wm2, bm2,
           edge_index, edge_type, batch):
    W1s = jnp.concatenate([W1, root1[None]], axis=0)
    W2s = jnp.concatenate([W2, root2[None]], axis=0)
    zeros_nd = jnp.zeros((N, D), jnp.float32)
    b1r = b1.reshape(1, D)
    b2r = b2.reshape(1, D)
    batch3 = batch.reshape(NBLK, 1, BN)
    Wm2p = jnp.pad(Wm2, ((0, 0), (0, D - OUT)))
    bm2p = jnp.pad(bm2, (0, D - OUT)).reshape(1, D)

    src = edge_index[0]
    dst = edge_index[1]
    cnt_parts = _k1a(dst, edge_type)
    inv = _k1c(cnt_parts.reshape(NW, CPT // 128, 128)).reshape(CPT)
    gidx, norm = _k1b(src, dst, edge_type, inv)

    xall1 = _k2(x, W1s)
    msg1 = _agg(xall1, gidx, norm, dst, zeros_nd)
    xall2 = _k4(msg1, xall1, b1r, W2s)
    msg2 = _agg(xall2, gidx, norm, dst, zeros_nd)
    emb, sums, cnts = _k6(msg2, xall2, b2r, batch3)
    logits_p, probs_p = _k7(sums, cnts, Wm1, bm1.reshape(1, D), Wm2p, bm2p)
    return (logits_p[:, :OUT], probs_p[:, :OUT], emb)
